# baseline (speedup 1.0000x reference)
import functools

import jax
import jax.numpy as jnp
from jax import lax
from jax.experimental import pallas as pl
from jax.experimental.pallas import tpu as pltpu
from jax.experimental.pallas import tpu_sc as plsc

_NC = 2
_NS = 16
_NW = _NC * _NS

_B, _S = 16384, 50
_FLAT = _B * _S
_D = 32
_BPW = _FLAT // _NW
_C = 1600
_NCHUNK = _BPW // _C

_mesh = plsc.VectorSubcoreMesh(core_axis_name="c", subcore_axis_name="s")


@functools.partial(
    pl.kernel,
    out_type=jax.ShapeDtypeStruct((_FLAT, _D), jnp.float32),
    mesh=_mesh,
    scratch_types=[
        pltpu.VMEM((_C,), jnp.int32),
        pltpu.VMEM((_C,), jnp.int32),
        pltpu.VMEM((_C, _D), jnp.float32),
        pltpu.VMEM((_C, _D), jnp.float32),
        pltpu.SemaphoreType.DMA,
        pltpu.SemaphoreType.DMA,
        pltpu.SemaphoreType.DMA,
    ],
    compiler_params=pltpu.CompilerParams(use_tc_tiling_on_sc=False),
)
def _gather_kernel(idx_hbm, table_hbm, out_hbm, idx0, idx1, rows0, rows1,
                   gsem, wsem0, wsem1):
    wid = lax.axis_index("s") * _NC + lax.axis_index("c")
    wbase = wid * _BPW
    idx_v = (idx0, idx1)
    rows_v = (rows0, rows1)
    wsem = (wsem0, wsem1)

    @pl.loop(0, _NCHUNK, step=2)
    def _pair(go):
        for b in range(2):
            g = go + b
            base = wbase + g * _C

            @pl.when(g >= 2)
            def _drain():
                pltpu.make_async_copy(
                    rows_v[b], out_hbm.at[pl.ds(wbase, _C)], wsem[b]).wait()

            pltpu.sync_copy(idx_hbm.at[pl.ds(base, _C)], idx_v[b])
            pltpu.async_copy(table_hbm.at[idx_v[b]], rows_v[b], gsem).wait()
            pltpu.async_copy(rows_v[b], out_hbm.at[pl.ds(base, _C)], wsem[b])

    for b in range(2):
        pltpu.make_async_copy(
            rows_v[b], out_hbm.at[pl.ds(wbase, _C)], wsem[b]).wait()


def kernel(inputs, embedding):
    flat = inputs.reshape(_FLAT)
    out = _gather_kernel(flat, embedding)
    return out.reshape(_B, _S, _D)


# speedup vs baseline: 1.0001x; 1.0001x over previous
import functools

import jax
import jax.numpy as jnp
from jax import lax
from jax.experimental import pallas as pl
from jax.experimental.pallas import tpu as pltpu
from jax.experimental.pallas import tpu_sc as plsc

_NC = 2
_NS = 16
_NW = _NC * _NS

_B, _S = 16384, 50
_FLAT = _B * _S
_D = 32
_BPW = _FLAT // _NW
_C = 1600
_NCHUNK = _BPW // _C
_K = 8
_CK = _C // _K

_mesh = plsc.VectorSubcoreMesh(core_axis_name="c", subcore_axis_name="s")


@functools.partial(
    pl.kernel,
    out_type=jax.ShapeDtypeStruct((_FLAT, _D), jnp.float32),
    mesh=_mesh,
    scratch_types=[
        pltpu.VMEM((_C,), jnp.int32),
        pltpu.VMEM((_C,), jnp.int32),
        pltpu.VMEM((_C, _D), jnp.float32),
        pltpu.VMEM((_C, _D), jnp.float32),
        pltpu.SemaphoreType.DMA,
        pltpu.SemaphoreType.DMA,
        pltpu.SemaphoreType.DMA,
    ],
    compiler_params=pltpu.CompilerParams(use_tc_tiling_on_sc=False),
)
def _gather_kernel(idx_hbm, table_hbm, out_hbm, idx0, idx1, rows0, rows1,
                   gsem, wsem0, wsem1):
    wid = lax.axis_index("s") * _NC + lax.axis_index("c")
    wbase = wid * _BPW
    idx_v = (idx0, idx1)
    rows_v = (rows0, rows1)
    wsem = (wsem0, wsem1)

    @pl.loop(0, _NCHUNK, step=2)
    def _pair(go):
        for b in range(2):
            g = go + b
            base = wbase + g * _C

            @pl.when(g >= 2)
            def _drain():
                pltpu.make_async_copy(
                    rows_v[b], out_hbm.at[pl.ds(wbase, _C)], wsem[b]).wait()

            pltpu.sync_copy(idx_hbm.at[pl.ds(base, _C)], idx_v[b])
            for j in range(_K):
                pltpu.async_copy(
                    table_hbm.at[idx_v[b].at[pl.ds(j * _CK, _CK)]],
                    rows_v[b].at[pl.ds(j * _CK, _CK)], gsem)
            for j in range(_K):
                pltpu.make_async_copy(
                    table_hbm.at[idx_v[b].at[pl.ds(0, _CK)]],
                    rows_v[b].at[pl.ds(0, _CK)], gsem).wait()
            pltpu.async_copy(rows_v[b], out_hbm.at[pl.ds(base, _C)], wsem[b])

    for b in range(2):
        pltpu.make_async_copy(
            rows_v[b], out_hbm.at[pl.ds(wbase, _C)], wsem[b]).wait()


def kernel(inputs, embedding):
    flat = inputs.reshape(_FLAT)
    out = _gather_kernel(flat, embedding)
    return out.reshape(_B, _S, _D)


# speedup vs baseline: 1.4546x; 1.4545x over previous
import functools

import jax
import jax.numpy as jnp
from jax import lax
from jax.experimental import pallas as pl
from jax.experimental.pallas import tpu as pltpu
from jax.experimental.pallas import tpu_sc as plsc

_NC = 2
_NS = 16
_NW = _NC * _NS

_B, _S = 16384, 50
_D = 32
_FLAT = _B * _S
_W = _B // _NW
_TBW = _W // 128

_mesh = plsc.VectorSubcoreMesh(core_axis_name="c", subcore_axis_name="s")


@functools.partial(
    pl.kernel,
    out_type=jax.ShapeDtypeStruct((_S * 4 * 128 * 8 * 128,), jnp.float32),
    mesh=_mesh,
    scratch_types=[
        pltpu.VMEM((_W,), jnp.int32),
        pltpu.VMEM((_W,), jnp.int32),
        pltpu.VMEM((_W, _D), jnp.float32),
        pltpu.VMEM((_W, _D), jnp.float32),
        pltpu.VMEM((_W * _D,), jnp.float32),
        pltpu.VMEM((_W * _D,), jnp.float32),
        pltpu.SemaphoreType.DMA,
        pltpu.SemaphoreType.DMA,
        pltpu.SemaphoreType.DMA,
        pltpu.SemaphoreType.DMA,
    ],
    compiler_params=pltpu.CompilerParams(use_tc_tiling_on_sc=False,
                                         needs_layout_passes=False),
)
def _gather_kernel(idx_hbm, table_hbm, out_hbm, idx0, idx1, rows0, rows1,
                   tblk0, tblk1, gsem0, gsem1, wsem0, wsem1):
    wid = lax.axis_index("s") * _NC + lax.axis_index("c")
    b0 = wid * _W
    idx_v = (idx0, idx1)
    rows_v = (rows0, rows1)
    tblk = (tblk0, tblk1)
    gsem = (gsem0, gsem1)
    wsem = (wsem0, wsem1)
    iota16 = lax.iota(jnp.int32, 16)
    zero16 = jnp.zeros((16,), jnp.int32)

    def load_and_fire(s, buf):
        pltpu.sync_copy(idx_hbm.at[pl.ds(s * _B + b0, _W)], idx_v[buf])
        pltpu.async_copy(table_hbm.at[idx_v[buf]], rows_v[buf], gsem[buf])

    def wait_gather(buf):
        pltpu.make_async_copy(
            table_hbm.at[idx_v[buf]], rows_v[buf], gsem[buf]).wait()

    def drain_writes(buf):
        pltpu.make_async_copy(
            tblk[buf], out_hbm.at[pl.ds(0, _W * _D)], wsem[buf]).wait()

    load_and_fire(0, 0)

    @pl.loop(0, _S, step=2)
    def _pair(go):
        for b in range(2):
            s = go + b
            nxt = (b + 1) % 2

            @pl.when(s >= 2)
            def _drain():
                drain_writes(b)

            @pl.when(s + 1 < _S)
            def _prefetch():
                load_and_fire(s + 1, nxt)

            wait_gather(b)

            @pl.loop(0, 16)
            def _tile(i):
                tf = i // 4
                tb = i % 4
                for sub in range(8):
                    col = zero16 + (tf * 8 + sub)
                    for l0 in range(0, 128, 16):
                        ridx = iota16 + (tb * 128 + l0)
                        g = plsc.load_gather(rows_v[b], [ridx, col])
                        tblk[b][pl.ds(((tf * 4 + tb) * 8 + sub) * 128 + l0,
                                      16)] = g

            for tf in range(4):
                off = ((s * 4 + tf) * 128 + wid * _TBW) * 1024
                pltpu.async_copy(
                    tblk[b].at[pl.ds(tf * 4096, 4096)],
                    out_hbm.at[pl.ds(off, 4096)], wsem[b])

    for b in range(2):
        drain_writes(b)


def kernel(inputs, embedding):
    idx_sm = inputs.T.reshape(_FLAT)
    out5f = _gather_kernel(idx_sm, embedding)
    out5 = out5f.reshape(_S, 4, 128, 8, 128)
    return out5.transpose(2, 4, 0, 1, 3).reshape(_B, _S, _D)


# speedup vs baseline: 2.3338x; 1.6044x over previous
import functools

import jax
import jax.numpy as jnp
from jax import lax
from jax.experimental import pallas as pl
from jax.experimental.pallas import tpu as pltpu
from jax.experimental.pallas import tpu_sc as plsc

_NC = 2
_NS = 16
_NW = _NC * _NS

_B, _S = 16384, 50
_D = 32
_W = _B // _NW
_TBW = _W // 128

_mesh = plsc.VectorSubcoreMesh(core_axis_name="c", subcore_axis_name="s")


@functools.partial(
    pl.kernel,
    out_type=jax.ShapeDtypeStruct((_S * 4 * 128 * 8 * 128,), jnp.float32),
    mesh=_mesh,
    scratch_types=[
        pltpu.VMEM((_S, _W), jnp.int32),
        pltpu.VMEM((_W, _D), jnp.float32),
        pltpu.VMEM((_W, _D), jnp.float32),
        pltpu.VMEM((_W * _D,), jnp.float32),
        pltpu.VMEM((_W * _D,), jnp.float32),
        pltpu.SemaphoreType.DMA,
        pltpu.SemaphoreType.DMA,
        pltpu.SemaphoreType.DMA,
        pltpu.SemaphoreType.DMA,
        pltpu.SemaphoreType.DMA,
    ],
    compiler_params=pltpu.CompilerParams(use_tc_tiling_on_sc=False,
                                         needs_layout_passes=False),
)
def _gather_kernel(idx_hbm, table_hbm, out_hbm, idxall, rows0, rows1,
                   tblk0, tblk1, isem, gsem0, gsem1, wsem0, wsem1):
    wid = lax.axis_index("s") * _NC + lax.axis_index("c")
    b0 = wid * _W
    rows_v = (rows0, rows1)
    tblk = (tblk0, tblk1)
    gsem = (gsem0, gsem1)
    wsem = (wsem0, wsem1)

    iota = lax.iota(jnp.int32, 16)
    skews = []
    for k in range(16):
        m = (iota + k) % 16
        skews.append((m, (m // 8) * 4096 + (m % 8) * 128 + iota))

    def fire_gather(s, buf):
        pltpu.async_copy(table_hbm.at[idxall.at[s]], rows_v[buf], gsem[buf])

    def wait_gather(buf):
        pltpu.make_async_copy(
            table_hbm.at[idxall.at[0]], rows_v[buf], gsem[buf]).wait()

    def drain_writes(buf):
        pltpu.make_async_copy(
            tblk[buf], out_hbm.at[pl.ds(0, _W * _D)], wsem[buf]).wait()

    pltpu.async_copy(idx_hbm.at[:, pl.ds(b0, _W)], idxall, isem).wait()
    fire_gather(0, 0)

    @pl.loop(0, _S, step=2)
    def _pair(go):
        for b in range(2):
            s = go + b
            nxt = (b + 1) % 2

            @pl.when(s >= 2)
            def _drain():
                drain_writes(b)

            @pl.when(s + 1 < _S)
            def _prefetch():
                fire_gather(s + 1, nxt)

            wait_gather(b)

            @pl.loop(0, _W // 16)
            def _rg(rg):
                r0 = rg * 16
                dbase_r = (r0 // 128) * 1024 + r0 % 128
                for c0 in (0, 16):
                    dbase = dbase_r + (c0 // 8) * 4096
                    for k in range(16):
                        m, dvec = skews[k]
                        g = plsc.load_gather(rows_v[b], [iota + r0, m + c0])
                        plsc.store_scatter(tblk[b], [dvec + dbase], g)

            for tf in range(4):
                off = ((s * 4 + tf) * 128 + wid * _TBW) * 1024
                pltpu.async_copy(
                    tblk[b].at[pl.ds(tf * 4096, 4096)],
                    out_hbm.at[pl.ds(off, 4096)], wsem[b])

    for b in range(2):
        drain_writes(b)


def kernel(inputs, embedding):
    out5f = _gather_kernel(inputs.T, embedding)
    out5 = out5f.reshape(_S, 4, 128, 8, 128)
    return out5.transpose(2, 4, 0, 1, 3).reshape(_B, _S, _D)


# speedup vs baseline: 2.3382x; 1.0019x over previous
import functools

import jax
import jax.numpy as jnp
from jax import lax
from jax.experimental import pallas as pl
from jax.experimental.pallas import tpu as pltpu
from jax.experimental.pallas import tpu_sc as plsc

_NC = 2
_NS = 16
_NW = _NC * _NS

_B, _S = 16384, 50
_D = 32
_W = _B // _NW
_TBW = _W // 128

_mesh = plsc.VectorSubcoreMesh(core_axis_name="c", subcore_axis_name="s")


@functools.partial(
    pl.kernel,
    out_type=jax.ShapeDtypeStruct((_S * 4 * 128 * 8 * 128,), jnp.float32),
    mesh=_mesh,
    scratch_types=[
        pltpu.VMEM((_S, _W), jnp.int32),
        pltpu.VMEM((_W, _D), jnp.float32),
        pltpu.VMEM((_W, _D), jnp.float32),
        pltpu.VMEM((_W * _D,), jnp.float32),
        pltpu.VMEM((_W * _D,), jnp.float32),
        pltpu.SemaphoreType.DMA,
        pltpu.SemaphoreType.DMA,
        pltpu.SemaphoreType.DMA,
        pltpu.SemaphoreType.DMA,
        pltpu.SemaphoreType.DMA,
    ],
    compiler_params=pltpu.CompilerParams(use_tc_tiling_on_sc=False,
                                         needs_layout_passes=False),
)
def _gather_kernel(idx_hbm, table_hbm, out_hbm, idxall, rows0, rows1,
                   tblk0, tblk1, isem, gsem0, gsem1, wsem0, wsem1):
    wid = lax.axis_index("s") * _NC + lax.axis_index("c")
    b0 = wid * _W
    rows_v = (rows0, rows1)
    tblk = (tblk0, tblk1)
    gsem = (gsem0, gsem1)
    wsem = (wsem0, wsem1)

    iota = lax.iota(jnp.int32, 16)
    skews = []
    for k in range(16):
        m = (iota + k) % 16
        skews.append((m, (m // 8) * 4096 + (m % 8) * 128 + iota))

    def fire_gather(s, buf):
        pltpu.async_copy(table_hbm.at[idxall.at[s]], rows_v[buf], gsem[buf])

    def wait_gather(buf):
        pltpu.make_async_copy(
            table_hbm.at[idxall.at[0]], rows_v[buf], gsem[buf]).wait()

    def drain_writes(buf):
        pltpu.make_async_copy(
            tblk[buf], out_hbm.at[pl.ds(0, _W * _D)], wsem[buf]).wait()

    pltpu.async_copy(idx_hbm.at[:, pl.ds(b0, _W)], idxall, isem).wait()
    fire_gather(0, 0)

    @pl.loop(0, _S, step=2)
    def _pair(go):
        for b in range(2):
            s = go + b
            nxt = (b + 1) % 2

            @pl.when(s >= 2)
            def _drain():
                drain_writes(b)

            @pl.when(s + 1 < _S)
            def _prefetch():
                fire_gather(s + 1, nxt)

            wait_gather(b)

            @pl.loop(0, _W // 16)
            def _rg(rg):
                r0 = rg * 16
                ridx = iota + r0
                dbase_r = (r0 // 128) * 1024 + r0 % 128
                for c0 in (0, 16):
                    dbase = dbase_r + (c0 // 8) * 4096
                    for k in range(16):
                        m, dvec = skews[k]
                        g = plsc.load_gather(rows_v[b], [ridx, m + c0])
                        plsc.store_scatter(tblk[b], [dvec + dbase], g)

            for tf in range(4):
                off = ((s * 4 + tf) * 128 + wid * _TBW) * 1024
                pltpu.async_copy(
                    tblk[b].at[pl.ds(tf * 4096, 4096)],
                    out_hbm.at[pl.ds(off, 4096)], wsem[b])

    for b in range(2):
        drain_writes(b)


def kernel(inputs, embedding):
    out5f = _gather_kernel(inputs.T, embedding)
    out5 = out5f.reshape(_S, 4, 128, 8, 128)
    return out5.transpose(2, 4, 0, 1, 3).reshape(_B, _S, _D)
